# baseline (speedup 1.0000x reference)
import jax
import jax.numpy as jnp
from jax import lax
from jax.experimental import pallas as pl
from jax.experimental.pallas import tpu as pltpu
from jax.experimental.pallas import tpu_sc as plsc

N = 10000
E = 320000
D = 128
NC = 2
NS = 16
CHUNK = 64
STEPS = 160
EPAD = NC * NS * STEPS * CHUNK
NPAD = 10016
CW = 16
RPT = NPAD // NS

NBUF = 4
NIDX = 8
GLA = 2
SDR = 2
ILA = 4


def _make_sc_body(with_counts):
    def body(feat_hbm, src_hbm, dst_hbm, aggp_hbm, *rest):
        if with_counts:
            (cntp_hbm, acc, cnt, src_idx, dst_idx, rows, ones_v, *sems) = rest
        else:
            (acc, cnt, src_idx, dst_idx, rows, ones_v, *sems) = rest
        sem_g = sems[0:NBUF]
        sem_s = sems[NBUF:2 * NBUF]
        sem_o = sems[2 * NBUF:3 * NBUF]
        sem_i = sems[3 * NBUF:3 * NBUF + NIDX]

        cid = lax.axis_index("c")
        sid = lax.axis_index("s")

        def zrow(i, _):
            r = i // (D // 16)
            c = i % (D // 16)
            rows[0, r, pl.ds(c * 16, 16)] = jnp.zeros((16,), jnp.float32)
            return 0
        lax.fori_loop(0, CHUNK * (D // 16), zrow, 0)

        row0 = sid * RPT

        def zacc(k, _):
            pltpu.sync_copy(rows.at[0], acc.at[pl.ds(row0 + k * CHUNK, CHUNK)])
            return 0
        lax.fori_loop(0, RPT // CHUNK, zacc, 0)
        rem = RPT % CHUNK
        if rem:
            pltpu.sync_copy(rows.at[0, pl.ds(0, rem)],
                            acc.at[pl.ds(row0 + (RPT // CHUNK) * CHUNK, rem)])

        if with_counts:
            def zones(i, _):
                ones_v[i, :] = jnp.zeros((CW,), jnp.float32)
                return 0
            lax.fori_loop(0, CHUNK, zones, 0)

            def zcnt2(k, _):
                pltpu.sync_copy(ones_v,
                                cnt.at[pl.ds(row0 + k * CHUNK, CHUNK)])
                return 0
            lax.fori_loop(0, RPT // CHUNK, zcnt2, 0)
            if rem:
                pltpu.sync_copy(
                    ones_v.at[pl.ds(0, rem)],
                    cnt.at[pl.ds(row0 + (RPT // CHUNK) * CHUNK, rem)])

            def onesfill(i, _):
                ones_v[i, :] = jnp.ones((CW,), jnp.float32)
                return 0
            lax.fori_loop(0, CHUNK, onesfill, 0)
        plsc.subcore_barrier()

        ebase = (cid * NS + sid) * (STEPS * CHUNK)

        def idx_load(j, s):
            pltpu.async_copy(
                src_hbm.at[pl.ds(ebase + j * CHUNK, CHUNK)],
                src_idx.at[s], sem_i[s])
            pltpu.async_copy(
                dst_hbm.at[pl.ds(ebase + j * CHUNK, CHUNK)],
                dst_idx.at[s], sem_i[s])

        def idx_wait(j, s):
            pltpu.make_async_copy(
                src_hbm.at[pl.ds(ebase + j * CHUNK, CHUNK)],
                src_idx.at[s], sem_i[s]).wait()
            pltpu.make_async_copy(
                dst_hbm.at[pl.ds(ebase + j * CHUNK, CHUNK)],
                dst_idx.at[s], sem_i[s]).wait()

        def gather_start(isl, rsl):
            pltpu.async_copy(feat_hbm.at[src_idx.at[isl]], rows.at[rsl],
                             sem_g[rsl])

        def gather_wait(isl, rsl):
            pltpu.make_async_copy(feat_hbm.at[src_idx.at[isl]], rows.at[rsl],
                                  sem_g[rsl]).wait()

        def scat_wait(isl, rsl):
            pltpu.make_async_copy(rows.at[rsl], acc.at[dst_idx.at[isl]],
                                  sem_s[rsl]).wait()

        def ones_wait(isl, rsl):
            pltpu.make_async_copy(ones_v, cnt.at[dst_idx.at[isl]],
                                  sem_o[rsl]).wait()

        for k in range(ILA):
            idx_load(k, k % NIDX)
        for k in range(GLA):
            idx_wait(k, k % NIDX)
            gather_start(k % NIDX, k % NBUF)

        def visit(j, v):
            rs = v % NBUF
            ns = (v + GLA) % NBUF
            is_j = v % NIDX
            is_g = (v + GLA) % NIDX
            is_d = (v + NIDX - SDR) % NIDX
            is_n = (v + ILA) % NIDX

            gather_wait(is_j, rs)
            pltpu.async_copy(rows.at[rs], acc.at[dst_idx.at[is_j]],
                             sem_s[rs], add=True)

            if with_counts:
                pltpu.async_copy(ones_v, cnt.at[dst_idx.at[is_j]],
                                 sem_o[rs], add=True)

            @pl.when(j >= SDR)
            def _():
                scat_wait(is_d, ns)
                if with_counts:
                    ones_wait(is_d, ns)

            @pl.when(j + ILA < STEPS)
            def _():
                idx_load(j + ILA, is_n)

            @pl.when(j + GLA < STEPS)
            def _():
                idx_wait(j + GLA, is_g)
                gather_start(is_g, ns)

        def outer(G, _):
            for v in range(NIDX):
                visit(NIDX * G + v, v)
            return 0
        lax.fori_loop(0, STEPS // NIDX, outer, 0)

        for k in range(STEPS - SDR, STEPS):
            scat_wait(k % NIDX, k % NBUF)
            if with_counts:
                ones_wait(k % NIDX, k % NBUF)

        plsc.subcore_barrier()

        pltpu.sync_copy(acc.at[pl.ds(row0, RPT)],
                        aggp_hbm.at[cid, pl.ds(row0, RPT)])
        if with_counts:
            pltpu.sync_copy(cnt.at[pl.ds(row0, RPT)],
                            cntp_hbm.at[cid, pl.ds(row0, RPT)])

    return body


def _make_sc_agg(with_counts):
    if with_counts:
        out_type = [jax.ShapeDtypeStruct((NC, NPAD, D), jnp.float32),
                    jax.ShapeDtypeStruct((NC, NPAD, CW), jnp.float32)]
    else:
        out_type = jax.ShapeDtypeStruct((NC, NPAD, D), jnp.float32)
    return pl.kernel(
        _make_sc_body(with_counts),
        out_type=out_type,
        mesh=plsc.VectorSubcoreMesh(core_axis_name="c",
                                    subcore_axis_name="s"),
        scratch_types=[
            pltpu.VMEM_SHARED((NPAD, D), jnp.float32),
            pltpu.VMEM_SHARED((NPAD, CW), jnp.float32),
            pltpu.VMEM((NIDX, CHUNK), jnp.int32),
            pltpu.VMEM((NIDX, CHUNK), jnp.int32),
            pltpu.VMEM((NBUF, CHUNK, D), jnp.float32),
            pltpu.VMEM((CHUNK, CW), jnp.float32),
        ] + [pltpu.SemaphoreType.DMA] * (3 * NBUF + NIDX),
        compiler_params=pltpu.CompilerParams(use_tc_tiling_on_sc=False),
    )


_sc_agg_cnt = _make_sc_agg(True)
_sc_agg_nocnt = _make_sc_agg(False)


R = 1000


def _tc_layer_body(aggp_ref, cntp_ref, x_ref, wlT_ref, wrT_ref, bl_ref,
                   whT_ref, bh_ref, h_ref, o_ref):
    agg = aggp_ref[0] + aggp_ref[1]
    cnt = cntp_ref[0, :, 0:1] + cntp_ref[1, :, 0:1]
    mean = agg * (1.0 / jnp.maximum(cnt, 1.0))
    h = (jnp.dot(mean, wlT_ref[...], preferred_element_type=jnp.float32)
         + jnp.dot(x_ref[...], wrT_ref[...], preferred_element_type=jnp.float32)
         + bl_ref[...])
    h = jnp.maximum(h, 0.0)
    h_ref[...] = h
    o_ref[...] = (jnp.dot(h, whT_ref[...], preferred_element_type=jnp.float32)
                  + bh_ref[...])


_tc_layer = pl.pallas_call(
    _tc_layer_body,
    grid=(N // R,),
    in_specs=[
        pl.BlockSpec((NC, R, D), lambda i: (0, i, 0)),
        pl.BlockSpec((NC, R, CW), lambda i: (0, i, 0)),
        pl.BlockSpec((R, D), lambda i: (i, 0)),
        pl.BlockSpec((D, D), lambda i: (0, 0)),
        pl.BlockSpec((D, D), lambda i: (0, 0)),
        pl.BlockSpec((1, D), lambda i: (0, 0)),
        pl.BlockSpec((D, 1), lambda i: (0, 0)),
        pl.BlockSpec((1, 1), lambda i: (0, 0)),
    ],
    out_specs=[
        pl.BlockSpec((R, D), lambda i: (i, 0)),
        pl.BlockSpec((R, 1), lambda i: (i, 0)),
    ],
    out_shape=[
        jax.ShapeDtypeStruct((N, D), jnp.float32),
        jax.ShapeDtypeStruct((N, 1), jnp.float32),
    ],
)


def kernel(x, edge_index, W1_l, b1_l, W1_r, W2_l, b2_l, W2_r, W_head, b_head):
    src = edge_index[0].astype(jnp.int32)
    dst = edge_index[1].astype(jnp.int32)
    npad = EPAD - E
    src = jnp.concatenate([src, jnp.zeros((npad,), jnp.int32)])
    dst = jnp.concatenate([dst, jnp.full((npad,), NPAD - 1, jnp.int32)])

    w_head_T = W_head.T
    b_head_c = b_head.reshape(1, 1)

    aggp1, cntp = _sc_agg_cnt(x, src, dst)
    h1, _ = _tc_layer(aggp1, cntp, x, W1_l.T, W1_r.T, b1_l.reshape(1, D),
                      w_head_T, b_head_c)
    aggp2 = _sc_agg_nocnt(h1, src, dst)
    h2, oc = _tc_layer(aggp2, cntp, h1, W2_l.T, W2_r.T, b2_l.reshape(1, D),
                       w_head_T, b_head_c)
    return (oc[:, 0], h2)


# speedup vs baseline: 1.4599x; 1.4599x over previous
import jax
import jax.numpy as jnp
from jax import lax
from jax.experimental import pallas as pl
from jax.experimental.pallas import tpu as pltpu
from jax.experimental.pallas import tpu_sc as plsc

N = 10000
E = 320000
D = 128
DH = D // 2
NC = 2
NS = 16
CHUNK = 128
STEPS = 160
EPAD = NS * STEPS * CHUNK
NPAD = 10240
CW = 16

NBUF = 4
NIDX = 8
GLA = 3
SDR = 1
ILA = 5


def _make_sc_body(with_counts):
    def body(feat_hbm, src_hbm, dst_hbm, aggp_hbm, *rest):
        if with_counts:
            (cntp_hbm, acc, cnt, src_idx, dst_idx, rows, ones_v, *sems) = rest
        else:
            (acc, cnt, src_idx, dst_idx, rows, ones_v, *sems) = rest
        sem_g = sems[0:NBUF]
        sem_s = sems[NBUF:2 * NBUF]
        sem_o = sems[2 * NBUF:3 * NBUF]
        sem_i = sems[3 * NBUF:3 * NBUF + NIDX]

        cid = lax.axis_index("c")
        sid = lax.axis_index("s")

        def zrow(i, _):
            r = i // (DH // 16)
            c = i % (DH // 16)
            rows[0, r, pl.ds(c * 16, 16)] = jnp.zeros((16,), jnp.float32)
            return 0
        lax.fori_loop(0, CHUNK * (DH // 16), zrow, 0)

        if with_counts:
            def zcnt(i, _):
                ones_v[i, :] = jnp.zeros((16,), jnp.float32)
                return 0
            lax.fori_loop(0, CHUNK, zcnt, 0)

        rows_per_tile = NPAD // NS
        row0 = sid * rows_per_tile

        def zacc(k, _):
            pltpu.sync_copy(rows.at[0], acc.at[pl.ds(row0 + k * CHUNK, CHUNK)])
            if with_counts:
                pltpu.sync_copy(ones_v,
                                cnt.at[pl.ds(row0 + k * CHUNK, CHUNK)])
            return 0
        lax.fori_loop(0, rows_per_tile // CHUNK, zacc, 0)

        if with_counts:
            def onesfill(i, _):
                ones_v[i, :] = jnp.ones((16,), jnp.float32)
                return 0
            lax.fori_loop(0, CHUNK, onesfill, 0)
        plsc.subcore_barrier()

        ebase = sid * (STEPS * CHUNK)

        def idx_load(j, s):
            pltpu.async_copy(
                src_hbm.at[cid, pl.ds(ebase + j * CHUNK, CHUNK)],
                src_idx.at[s], sem_i[s])
            pltpu.async_copy(
                dst_hbm.at[pl.ds(ebase + j * CHUNK, CHUNK)],
                dst_idx.at[s], sem_i[s])

        def idx_wait(j, s):
            pltpu.make_async_copy(
                src_hbm.at[cid, pl.ds(ebase + j * CHUNK, CHUNK)],
                src_idx.at[s], sem_i[s]).wait()
            pltpu.make_async_copy(
                dst_hbm.at[pl.ds(ebase + j * CHUNK, CHUNK)],
                dst_idx.at[s], sem_i[s]).wait()

        def gather_start(isl, rsl):
            pltpu.async_copy(feat_hbm.at[src_idx.at[isl]], rows.at[rsl],
                             sem_g[rsl])

        def gather_wait(isl, rsl):
            pltpu.make_async_copy(feat_hbm.at[src_idx.at[isl]], rows.at[rsl],
                                  sem_g[rsl]).wait()

        def scat_wait(isl, rsl):
            pltpu.make_async_copy(rows.at[rsl], acc.at[dst_idx.at[isl]],
                                  sem_s[rsl]).wait()

        def ones_wait(isl, rsl):
            pltpu.make_async_copy(ones_v, cnt.at[dst_idx.at[isl]],
                                  sem_o[rsl]).wait()

        for k in range(ILA):
            idx_load(k, k % NIDX)
        for k in range(GLA):
            idx_wait(k, k % NIDX)
            gather_start(k % NIDX, k % NBUF)

        def visit(j, v):
            rs = v % NBUF
            ns = (v + GLA) % NBUF
            is_j = v % NIDX
            is_g = (v + GLA) % NIDX
            is_d = (v + NIDX - SDR) % NIDX
            is_n = (v + ILA) % NIDX

            gather_wait(is_j, rs)
            pltpu.async_copy(rows.at[rs], acc.at[dst_idx.at[is_j]],
                             sem_s[rs], add=True)

            if with_counts:
                @pl.when(cid == v % 2)
                def _():
                    pltpu.async_copy(ones_v, cnt.at[dst_idx.at[is_j]],
                                     sem_o[rs], add=True)

            @pl.when(j >= SDR)
            def _():
                scat_wait(is_d, ns)

                if with_counts:
                    @pl.when(cid == (v + SDR) % 2)
                    def _():
                        ones_wait(is_d, ns)

            @pl.when(j + ILA < STEPS)
            def _():
                idx_load(j + ILA, is_n)

            @pl.when(j + GLA < STEPS)
            def _():
                idx_wait(j + GLA, is_g)
                gather_start(is_g, ns)

        def outer(G, _):
            for v in range(NIDX):
                visit(NIDX * G + v, v)
            return 0
        lax.fori_loop(0, STEPS // NIDX, outer, 0)

        for k in range(STEPS - SDR, STEPS):
            scat_wait(k % NIDX, k % NBUF)
            if with_counts:
                @pl.when(cid == k % 2)
                def _():
                    ones_wait(k % NIDX, k % NBUF)

        plsc.subcore_barrier()

        pltpu.sync_copy(acc.at[pl.ds(row0, rows_per_tile)],
                        aggp_hbm.at[cid, pl.ds(row0, rows_per_tile)])
        if with_counts:
            pltpu.sync_copy(cnt.at[pl.ds(row0, rows_per_tile)],
                            cntp_hbm.at[cid, pl.ds(row0, rows_per_tile)])

    return body


def _make_sc_agg(with_counts):
    if with_counts:
        out_type = [jax.ShapeDtypeStruct((NC, NPAD, DH), jnp.float32),
                    jax.ShapeDtypeStruct((NC, NPAD, CW), jnp.float32)]
    else:
        out_type = jax.ShapeDtypeStruct((NC, NPAD, DH), jnp.float32)
    return pl.kernel(
        _make_sc_body(with_counts),
        out_type=out_type,
        mesh=plsc.VectorSubcoreMesh(core_axis_name="c",
                                    subcore_axis_name="s"),
        scratch_types=[
            pltpu.VMEM_SHARED((NPAD, DH), jnp.float32),
            pltpu.VMEM_SHARED((NPAD, CW), jnp.float32),
            pltpu.VMEM((NIDX, CHUNK), jnp.int32),
            pltpu.VMEM((NIDX, CHUNK), jnp.int32),
            pltpu.VMEM((NBUF, CHUNK, DH), jnp.float32),
            pltpu.VMEM((CHUNK, CW), jnp.float32),
        ] + [pltpu.SemaphoreType.DMA] * (3 * NBUF + NIDX),
        compiler_params=pltpu.CompilerParams(use_tc_tiling_on_sc=False),
    )


_sc_agg_cnt = _make_sc_agg(True)
_sc_agg_nocnt = _make_sc_agg(False)


R = 1000


def _tc_layer_body(aggp_ref, cntp_ref, x_ref, wlT_ref, wrT_ref, bl_ref,
                   whT_ref, bh_ref, h_ref, hs_ref, o_ref):
    agg = jnp.concatenate([aggp_ref[0], aggp_ref[1]], axis=1)
    cnt = cntp_ref[0, :, 0:1] + cntp_ref[1, :, 0:1]
    mean = agg * (1.0 / jnp.maximum(cnt, 1.0))
    h = (jnp.dot(mean, wlT_ref[...], preferred_element_type=jnp.float32)
         + jnp.dot(x_ref[...], wrT_ref[...], preferred_element_type=jnp.float32)
         + bl_ref[...])
    h = jnp.maximum(h, 0.0)
    h_ref[...] = h
    hs_ref[0] = h[:, :DH]
    hs_ref[1] = h[:, DH:]
    o_ref[...] = (jnp.dot(h, whT_ref[...], preferred_element_type=jnp.float32)
                  + bh_ref[...])


_tc_layer = pl.pallas_call(
    _tc_layer_body,
    grid=(N // R,),
    in_specs=[
        pl.BlockSpec((NC, R, DH), lambda i: (0, i, 0)),
        pl.BlockSpec((NC, R, CW), lambda i: (0, i, 0)),
        pl.BlockSpec((R, D), lambda i: (i, 0)),
        pl.BlockSpec((D, D), lambda i: (0, 0)),
        pl.BlockSpec((D, D), lambda i: (0, 0)),
        pl.BlockSpec((1, D), lambda i: (0, 0)),
        pl.BlockSpec((D, 1), lambda i: (0, 0)),
        pl.BlockSpec((1, 1), lambda i: (0, 0)),
    ],
    out_specs=[
        pl.BlockSpec((R, D), lambda i: (i, 0)),
        pl.BlockSpec((NC, R, DH), lambda i: (0, i, 0)),
        pl.BlockSpec((R, 1), lambda i: (i, 0)),
    ],
    out_shape=[
        jax.ShapeDtypeStruct((N, D), jnp.float32),
        jax.ShapeDtypeStruct((NC, N, DH), jnp.float32),
        jax.ShapeDtypeStruct((N, 1), jnp.float32),
    ],
)


def kernel(x, edge_index, W1_l, b1_l, W1_r, W2_l, b2_l, W2_r, W_head, b_head):
    src = edge_index[0].astype(jnp.int32)
    dst = edge_index[1].astype(jnp.int32)
    npad = EPAD - E
    src = jnp.concatenate([src, jnp.zeros((npad,), jnp.int32)])
    dst = jnp.concatenate([dst, jnp.full((npad,), NPAD - 1, jnp.int32)])
    src_stk = jnp.stack([src, src + N])

    xs = jnp.concatenate([x[:, :DH], x[:, DH:]], axis=0)

    w_head_T = W_head.T
    b_head_c = b_head.reshape(1, 1)

    aggp1, cntp = _sc_agg_cnt(xs, src_stk, dst)
    h1, h1s, _ = _tc_layer(aggp1, cntp, x, W1_l.T, W1_r.T, b1_l.reshape(1, D),
                           w_head_T, b_head_c)
    aggp2 = _sc_agg_nocnt(h1s.reshape(NC * N, DH), src_stk, dst)
    h2, _, oc = _tc_layer(aggp2, cntp, h1, W2_l.T, W2_r.T, b2_l.reshape(1, D),
                          w_head_T, b_head_c)
    return (oc[:, 0], h2)


# speedup vs baseline: 1.9184x; 1.3140x over previous
import jax
import jax.numpy as jnp
from jax import lax
from jax.experimental import pallas as pl
from jax.experimental.pallas import tpu as pltpu
from jax.experimental.pallas import tpu_sc as plsc

N = 10000
E = 320000
D = 128
DH = D // 2
NC = 2
NS = 16
CHUNK = 128
STEPS = 160
EPAD = NS * STEPS * CHUNK
NPAD = 10240
CW = 16

NBUF = 4
NIDX = 8
GLA = 3
SDR = 1
ILA = 5


def _make_sc_body(with_counts):
    use_table = not with_counts

    def body(feat_hbm, src_hbm, dst_hbm, aggp_hbm, *rest):
        if with_counts:
            (cntp_hbm, acc, cnt, src_idx, dst_idx, rows, ones_v, *sems) = rest
        else:
            (acc, table, src_idx, dst_idx, rows, ones_v, *sems) = rest
        sem_g = sems[0:NBUF]
        sem_s = sems[NBUF:2 * NBUF]
        sem_o = sems[2 * NBUF:3 * NBUF]
        sem_i = sems[3 * NBUF:3 * NBUF + NIDX]

        cid = lax.axis_index("c")
        sid = lax.axis_index("s")

        def zrow(i, _):
            r = i // (DH // 16)
            c = i % (DH // 16)
            rows[0, r, pl.ds(c * 16, 16)] = jnp.zeros((16,), jnp.float32)
            return 0
        lax.fori_loop(0, CHUNK * (DH // 16), zrow, 0)

        if with_counts:
            def zcnt(i, _):
                ones_v[i, :] = jnp.zeros((16,), jnp.float32)
                return 0
            lax.fori_loop(0, CHUNK, zcnt, 0)

        rows_per_tile = NPAD // NS
        row0 = sid * rows_per_tile

        def zacc(k, _):
            pltpu.sync_copy(rows.at[0], acc.at[pl.ds(row0 + k * CHUNK, CHUNK)])
            if with_counts:
                pltpu.sync_copy(ones_v,
                                cnt.at[pl.ds(row0 + k * CHUNK, CHUNK)])
            return 0
        lax.fori_loop(0, rows_per_tile // CHUNK, zacc, 0)

        if with_counts:
            def onesfill(i, _):
                ones_v[i, :] = jnp.ones((16,), jnp.float32)
                return 0
            lax.fori_loop(0, CHUNK, onesfill, 0)

        if use_table:
            trows = N // NS
            pltpu.sync_copy(
                feat_hbm.at[pl.ds(cid * N + sid * trows, trows)],
                table.at[pl.ds(sid * trows, trows)])
        plsc.subcore_barrier()

        ebase = sid * (STEPS * CHUNK)

        def src_slice(j):
            if use_table:
                return src_hbm.at[pl.ds(ebase + j * CHUNK, CHUNK)]
            return src_hbm.at[cid, pl.ds(ebase + j * CHUNK, CHUNK)]

        def idx_load(j, s):
            pltpu.async_copy(src_slice(j), src_idx.at[s], sem_i[s])
            pltpu.async_copy(
                dst_hbm.at[pl.ds(ebase + j * CHUNK, CHUNK)],
                dst_idx.at[s], sem_i[s])

        def idx_wait(j, s):
            pltpu.make_async_copy(src_slice(j), src_idx.at[s],
                                  sem_i[s]).wait()
            pltpu.make_async_copy(
                dst_hbm.at[pl.ds(ebase + j * CHUNK, CHUNK)],
                dst_idx.at[s], sem_i[s]).wait()

        gsrc = table if use_table else feat_hbm

        def gather_start(isl, rsl):
            pltpu.async_copy(gsrc.at[src_idx.at[isl]], rows.at[rsl],
                             sem_g[rsl])

        def gather_wait(isl, rsl):
            pltpu.make_async_copy(gsrc.at[src_idx.at[isl]], rows.at[rsl],
                                  sem_g[rsl]).wait()

        def scat_wait(isl, rsl):
            pltpu.make_async_copy(rows.at[rsl], acc.at[dst_idx.at[isl]],
                                  sem_s[rsl]).wait()

        def ones_wait(isl, rsl):
            pltpu.make_async_copy(ones_v, cnt.at[dst_idx.at[isl]],
                                  sem_o[rsl]).wait()

        for k in range(ILA):
            idx_load(k, k % NIDX)
        for k in range(GLA):
            idx_wait(k, k % NIDX)
            gather_start(k % NIDX, k % NBUF)

        def visit(j, v):
            rs = v % NBUF
            ns = (v + GLA) % NBUF
            is_j = v % NIDX
            is_g = (v + GLA) % NIDX
            is_d = (v + NIDX - SDR) % NIDX
            is_n = (v + ILA) % NIDX

            gather_wait(is_j, rs)
            pltpu.async_copy(rows.at[rs], acc.at[dst_idx.at[is_j]],
                             sem_s[rs], add=True)

            if with_counts:
                @pl.when(cid == v % 2)
                def _():
                    pltpu.async_copy(ones_v, cnt.at[dst_idx.at[is_j]],
                                     sem_o[rs], add=True)

            @pl.when(j >= SDR)
            def _():
                scat_wait(is_d, ns)

                if with_counts:
                    @pl.when(cid == (v + SDR) % 2)
                    def _():
                        ones_wait(is_d, ns)

            @pl.when(j + ILA < STEPS)
            def _():
                idx_load(j + ILA, is_n)

            @pl.when(j + GLA < STEPS)
            def _():
                idx_wait(j + GLA, is_g)
                gather_start(is_g, ns)

        def outer(G, _):
            for v in range(NIDX):
                visit(NIDX * G + v, v)
            return 0
        lax.fori_loop(0, STEPS // NIDX, outer, 0)

        for k in range(STEPS - SDR, STEPS):
            scat_wait(k % NIDX, k % NBUF)
            if with_counts:
                @pl.when(cid == k % 2)
                def _():
                    ones_wait(k % NIDX, k % NBUF)

        plsc.subcore_barrier()

        pltpu.sync_copy(acc.at[pl.ds(row0, rows_per_tile)],
                        aggp_hbm.at[cid, pl.ds(row0, rows_per_tile)])
        if with_counts:
            pltpu.sync_copy(cnt.at[pl.ds(row0, rows_per_tile)],
                            cntp_hbm.at[cid, pl.ds(row0, rows_per_tile)])

    return body


def _make_sc_agg(with_counts):
    if with_counts:
        out_type = [jax.ShapeDtypeStruct((NC, NPAD, DH), jnp.float32),
                    jax.ShapeDtypeStruct((NC, NPAD, CW), jnp.float32)]
    else:
        out_type = jax.ShapeDtypeStruct((NC, NPAD, DH), jnp.float32)
    return pl.kernel(
        _make_sc_body(with_counts),
        out_type=out_type,
        mesh=plsc.VectorSubcoreMesh(core_axis_name="c",
                                    subcore_axis_name="s"),
        scratch_types=[
            pltpu.VMEM_SHARED((NPAD, DH), jnp.float32),
            pltpu.VMEM_SHARED((NPAD, CW), jnp.float32) if with_counts
            else pltpu.VMEM_SHARED((N, DH), jnp.float32),
            pltpu.VMEM((NIDX, CHUNK), jnp.int32),
            pltpu.VMEM((NIDX, CHUNK), jnp.int32),
            pltpu.VMEM((NBUF, CHUNK, DH), jnp.float32),
            pltpu.VMEM((CHUNK, CW), jnp.float32),
        ] + [pltpu.SemaphoreType.DMA] * (3 * NBUF + NIDX),
        compiler_params=pltpu.CompilerParams(use_tc_tiling_on_sc=False),
    )


_sc_agg_cnt = _make_sc_agg(True)
_sc_agg_nocnt = _make_sc_agg(False)


R = 1000


def _tc_layer_body(aggp_ref, cntp_ref, x_ref, wlT_ref, wrT_ref, bl_ref,
                   whT_ref, bh_ref, h_ref, hs_ref, o_ref):
    agg = jnp.concatenate([aggp_ref[0], aggp_ref[1]], axis=1)
    cnt = cntp_ref[0, :, 0:1] + cntp_ref[1, :, 0:1]
    mean = agg * (1.0 / jnp.maximum(cnt, 1.0))
    h = (jnp.dot(mean, wlT_ref[...], preferred_element_type=jnp.float32)
         + jnp.dot(x_ref[...], wrT_ref[...], preferred_element_type=jnp.float32)
         + bl_ref[...])
    h = jnp.maximum(h, 0.0)
    h_ref[...] = h
    hs_ref[0] = h[:, :DH]
    hs_ref[1] = h[:, DH:]
    o_ref[...] = (jnp.dot(h, whT_ref[...], preferred_element_type=jnp.float32)
                  + bh_ref[...])


_tc_layer = pl.pallas_call(
    _tc_layer_body,
    grid=(N // R,),
    in_specs=[
        pl.BlockSpec((NC, R, DH), lambda i: (0, i, 0)),
        pl.BlockSpec((NC, R, CW), lambda i: (0, i, 0)),
        pl.BlockSpec((R, D), lambda i: (i, 0)),
        pl.BlockSpec((D, D), lambda i: (0, 0)),
        pl.BlockSpec((D, D), lambda i: (0, 0)),
        pl.BlockSpec((1, D), lambda i: (0, 0)),
        pl.BlockSpec((D, 1), lambda i: (0, 0)),
        pl.BlockSpec((1, 1), lambda i: (0, 0)),
    ],
    out_specs=[
        pl.BlockSpec((R, D), lambda i: (i, 0)),
        pl.BlockSpec((NC, R, DH), lambda i: (0, i, 0)),
        pl.BlockSpec((R, 1), lambda i: (i, 0)),
    ],
    out_shape=[
        jax.ShapeDtypeStruct((N, D), jnp.float32),
        jax.ShapeDtypeStruct((NC, N, DH), jnp.float32),
        jax.ShapeDtypeStruct((N, 1), jnp.float32),
    ],
)


def kernel(x, edge_index, W1_l, b1_l, W1_r, W2_l, b2_l, W2_r, W_head, b_head):
    src = edge_index[0].astype(jnp.int32)
    dst = edge_index[1].astype(jnp.int32)
    npad = EPAD - E
    src = jnp.concatenate([src, jnp.zeros((npad,), jnp.int32)])
    dst = jnp.concatenate([dst, jnp.full((npad,), NPAD - 1, jnp.int32)])
    src_stk = jnp.stack([src, src + N])

    xs = jnp.concatenate([x[:, :DH], x[:, DH:]], axis=0)

    w_head_T = W_head.T
    b_head_c = b_head.reshape(1, 1)

    aggp1, cntp = _sc_agg_cnt(xs, src_stk, dst)
    h1, h1s, _ = _tc_layer(aggp1, cntp, x, W1_l.T, W1_r.T, b1_l.reshape(1, D),
                           w_head_T, b_head_c)
    aggp2 = _sc_agg_nocnt(h1s.reshape(NC * N, DH), src, dst)
    h2, _, oc = _tc_layer(aggp2, cntp, h1, W2_l.T, W2_r.T, b2_l.reshape(1, D),
                          w_head_T, b_head_c)
    return (oc[:, 0], h2)


# speedup vs baseline: 2.6446x; 1.3785x over previous
import jax
import jax.numpy as jnp
from jax import lax
from jax.experimental import pallas as pl
from jax.experimental.pallas import tpu as pltpu
from jax.experimental.pallas import tpu_sc as plsc

N = 10000
E = 320000
D = 128
DH = D // 2
NC = 2
NS = 16
NPAD = 10240
CW = 16

NBUF = 4
NIDX = 8
GLA = 3
SDR = 1
ILA = 5

CHUNK1, STEPS1 = 96, 216
CHUNK2, STEPS2 = 128, 160
EPAD1 = NS * STEPS1 * CHUNK1
EPAD2 = NS * STEPS2 * CHUNK2


def _make_sc_body(with_counts, CHUNK, STEPS):
    def body(feat_hbm, src_hbm, dst_hbm, aggp_hbm, *rest):
        if with_counts:
            (cntp_hbm, acc, table, cnt, src_idx, dst_idx, rows, ones_v,
             *sems) = rest
        else:
            (acc, table, src_idx, dst_idx, rows, ones_v, *sems) = rest
        sem_g = sems[0:NBUF]
        sem_s = sems[NBUF:2 * NBUF]
        sem_o = sems[2 * NBUF:3 * NBUF]
        sem_i = sems[3 * NBUF:3 * NBUF + NIDX]

        cid = lax.axis_index("c")
        sid = lax.axis_index("s")

        def zrow(i, _):
            r = i // (DH // 16)
            c = i % (DH // 16)
            rows[0, r, pl.ds(c * 16, 16)] = jnp.zeros((16,), jnp.float32)
            return 0
        lax.fori_loop(0, CHUNK * (DH // 16), zrow, 0)

        if with_counts:
            def zcnt(i, _):
                ones_v[i, :] = jnp.zeros((16,), jnp.float32)
                return 0
            lax.fori_loop(0, CHUNK, zcnt, 0)

        rows_per_tile = NPAD // NS
        row0 = sid * rows_per_tile
        nfull = rows_per_tile // CHUNK
        rem = rows_per_tile % CHUNK

        def zacc(k, _):
            pltpu.sync_copy(rows.at[0], acc.at[pl.ds(row0 + k * CHUNK, CHUNK)])
            if with_counts:
                pltpu.sync_copy(ones_v,
                                cnt.at[pl.ds(row0 + k * CHUNK, CHUNK)])
            return 0
        lax.fori_loop(0, nfull, zacc, 0)
        if rem:
            pltpu.sync_copy(rows.at[0, pl.ds(0, rem)],
                            acc.at[pl.ds(row0 + nfull * CHUNK, rem)])
            if with_counts:
                pltpu.sync_copy(ones_v.at[pl.ds(0, rem)],
                                cnt.at[pl.ds(row0 + nfull * CHUNK, rem)])

        if with_counts:
            def onesfill(i, _):
                ones_v[i, :] = jnp.ones((16,), jnp.float32)
                return 0
            lax.fori_loop(0, CHUNK, onesfill, 0)

        trows = N // NS
        pltpu.sync_copy(
            feat_hbm.at[pl.ds(cid * N + sid * trows, trows)],
            table.at[pl.ds(sid * trows, trows)])
        plsc.subcore_barrier()

        ebase = sid * (STEPS * CHUNK)

        def src_slice(j):
            return src_hbm.at[pl.ds(ebase + j * CHUNK, CHUNK)]

        def dst_slice(j):
            return dst_hbm.at[pl.ds(ebase + j * CHUNK, CHUNK)]

        def idx_load(j, s):
            pltpu.async_copy(src_slice(j), src_idx.at[s], sem_i[s])
            pltpu.async_copy(dst_slice(j), dst_idx.at[s], sem_i[s])

        def idx_wait(j, s):
            pltpu.make_async_copy(src_slice(j), src_idx.at[s],
                                  sem_i[s]).wait()
            pltpu.make_async_copy(dst_slice(j), dst_idx.at[s],
                                  sem_i[s]).wait()

        def gather_start(isl, rsl):
            pltpu.async_copy(table.at[src_idx.at[isl]], rows.at[rsl],
                             sem_g[rsl])

        def gather_wait(isl, rsl):
            pltpu.make_async_copy(table.at[src_idx.at[isl]], rows.at[rsl],
                                  sem_g[rsl]).wait()

        def scat_wait(isl, rsl):
            pltpu.make_async_copy(rows.at[rsl], acc.at[dst_idx.at[isl]],
                                  sem_s[rsl]).wait()

        def ones_wait(isl, rsl):
            pltpu.make_async_copy(ones_v, cnt.at[dst_idx.at[isl]],
                                  sem_o[rsl]).wait()

        for k in range(ILA):
            idx_load(k, k % NIDX)
        for k in range(GLA):
            idx_wait(k, k % NIDX)
            gather_start(k % NIDX, k % NBUF)

        def visit(j, v):
            rs = v % NBUF
            ns = (v + GLA) % NBUF
            is_j = v % NIDX
            is_g = (v + GLA) % NIDX
            is_d = (v + NIDX - SDR) % NIDX
            is_n = (v + ILA) % NIDX

            gather_wait(is_j, rs)
            pltpu.async_copy(rows.at[rs], acc.at[dst_idx.at[is_j]],
                             sem_s[rs], add=True)

            if with_counts:
                @pl.when(cid == v % 2)
                def _():
                    pltpu.async_copy(ones_v, cnt.at[dst_idx.at[is_j]],
                                     sem_o[rs], add=True)

            @pl.when(j >= SDR)
            def _():
                scat_wait(is_d, ns)

                if with_counts:
                    @pl.when(cid == (v + SDR) % 2)
                    def _():
                        ones_wait(is_d, ns)

            @pl.when(j + ILA < STEPS)
            def _():
                idx_load(j + ILA, is_n)

            @pl.when(j + GLA < STEPS)
            def _():
                idx_wait(j + GLA, is_g)
                gather_start(is_g, ns)

        def outer(G, _):
            for v in range(NIDX):
                visit(NIDX * G + v, v)
            return 0
        lax.fori_loop(0, STEPS // NIDX, outer, 0)

        for k in range(STEPS - SDR, STEPS):
            scat_wait(k % NIDX, k % NBUF)
            if with_counts:
                @pl.when(cid == k % 2)
                def _():
                    ones_wait(k % NIDX, k % NBUF)

        plsc.subcore_barrier()

        pltpu.sync_copy(acc.at[pl.ds(row0, rows_per_tile)],
                        aggp_hbm.at[cid, pl.ds(row0, rows_per_tile)])
        if with_counts:
            pltpu.sync_copy(cnt.at[pl.ds(row0, rows_per_tile)],
                            cntp_hbm.at[cid, pl.ds(row0, rows_per_tile)])

    return body


def _make_sc_agg(with_counts, CHUNK, STEPS):
    if with_counts:
        out_type = [jax.ShapeDtypeStruct((NC, NPAD, DH), jnp.float32),
                    jax.ShapeDtypeStruct((NC, NPAD, CW), jnp.float32)]
    else:
        out_type = jax.ShapeDtypeStruct((NC, NPAD, DH), jnp.float32)
    scratch = [
        pltpu.VMEM_SHARED((NPAD, DH), jnp.float32),
        pltpu.VMEM_SHARED((N, DH), jnp.float32),
    ]
    if with_counts:
        scratch.append(pltpu.VMEM_SHARED((NPAD, CW), jnp.float32))
    scratch += [
        pltpu.VMEM((NIDX, CHUNK), jnp.int32),
        pltpu.VMEM((NIDX, CHUNK), jnp.int32),
        pltpu.VMEM((NBUF, CHUNK, DH), jnp.float32),
        pltpu.VMEM((CHUNK, CW), jnp.float32),
    ] + [pltpu.SemaphoreType.DMA] * (3 * NBUF + NIDX)
    return pl.kernel(
        _make_sc_body(with_counts, CHUNK, STEPS),
        out_type=out_type,
        mesh=plsc.VectorSubcoreMesh(core_axis_name="c",
                                    subcore_axis_name="s"),
        scratch_types=scratch,
        compiler_params=pltpu.CompilerParams(use_tc_tiling_on_sc=False),
    )


_sc_agg_cnt = _make_sc_agg(True, CHUNK1, STEPS1)
_sc_agg_nocnt = _make_sc_agg(False, CHUNK2, STEPS2)


R = 1000


def _tc_layer_body(aggp_ref, cntp_ref, x_ref, wlT_ref, wrT_ref, bl_ref,
                   whT_ref, bh_ref, h_ref, hs_ref, o_ref):
    agg = jnp.concatenate([aggp_ref[0], aggp_ref[1]], axis=1)
    cnt = cntp_ref[0, :, 0:1] + cntp_ref[1, :, 0:1]
    mean = agg * (1.0 / jnp.maximum(cnt, 1.0))
    h = (jnp.dot(mean, wlT_ref[...], preferred_element_type=jnp.float32)
         + jnp.dot(x_ref[...], wrT_ref[...], preferred_element_type=jnp.float32)
         + bl_ref[...])
    h = jnp.maximum(h, 0.0)
    h_ref[...] = h
    hs_ref[0] = h[:, :DH]
    hs_ref[1] = h[:, DH:]
    o_ref[...] = (jnp.dot(h, whT_ref[...], preferred_element_type=jnp.float32)
                  + bh_ref[...])


_tc_layer = pl.pallas_call(
    _tc_layer_body,
    grid=(N // R,),
    in_specs=[
        pl.BlockSpec((NC, R, DH), lambda i: (0, i, 0)),
        pl.BlockSpec((NC, R, CW), lambda i: (0, i, 0)),
        pl.BlockSpec((R, D), lambda i: (i, 0)),
        pl.BlockSpec((D, D), lambda i: (0, 0)),
        pl.BlockSpec((D, D), lambda i: (0, 0)),
        pl.BlockSpec((1, D), lambda i: (0, 0)),
        pl.BlockSpec((D, 1), lambda i: (0, 0)),
        pl.BlockSpec((1, 1), lambda i: (0, 0)),
    ],
    out_specs=[
        pl.BlockSpec((R, D), lambda i: (i, 0)),
        pl.BlockSpec((NC, R, DH), lambda i: (0, i, 0)),
        pl.BlockSpec((R, 1), lambda i: (i, 0)),
    ],
    out_shape=[
        jax.ShapeDtypeStruct((N, D), jnp.float32),
        jax.ShapeDtypeStruct((NC, N, DH), jnp.float32),
        jax.ShapeDtypeStruct((N, 1), jnp.float32),
    ],
)


def kernel(x, edge_index, W1_l, b1_l, W1_r, W2_l, b2_l, W2_r, W_head, b_head):
    src = edge_index[0].astype(jnp.int32)
    dst = edge_index[1].astype(jnp.int32)

    def pad_to(a, n, fill):
        return jnp.concatenate([a, jnp.full((n - E,), fill, jnp.int32)])

    src1 = pad_to(src, EPAD1, 0)
    dst1 = pad_to(dst, EPAD1, NPAD - 1)
    src2 = pad_to(src, EPAD2, 0)
    dst2 = pad_to(dst, EPAD2, NPAD - 1)

    xs = jnp.concatenate([x[:, :DH], x[:, DH:]], axis=0)

    w_head_T = W_head.T
    b_head_c = b_head.reshape(1, 1)

    aggp1, cntp = _sc_agg_cnt(xs, src1, dst1)
    h1, h1s, _ = _tc_layer(aggp1, cntp, x, W1_l.T, W1_r.T, b1_l.reshape(1, D),
                           w_head_T, b_head_c)
    aggp2 = _sc_agg_nocnt(h1s.reshape(NC * N, DH), src2, dst2)
    h2, _, oc = _tc_layer(aggp2, cntp, h1, W2_l.T, W2_r.T, b2_l.reshape(1, D),
                          w_head_T, b_head_c)
    return (oc[:, 0], h2)
